# Initial kernel scaffold; baseline (speedup 1.0000x reference)
import jax
import jax.numpy as jnp
from jax.experimental import pallas as pl


def kernel(loc_data, conf_data, prior_data, conf_thresh, nms_thresh):
    raise NotImplementedError("write your pallas kernel here")


# speedup vs baseline: 3.5131x; 3.5131x over previous
import functools

import jax
import jax.numpy as jnp
from jax.experimental import pallas as pl
from jax.experimental.pallas import tpu as pltpu

_VAR0 = 0.1
_VAR1 = 0.2
_TOP_K = 200
_P = 20000
_PP = 20480
_ROWS = 160
_LANES = 128
_NEG = -jnp.inf


def _prep_kernel(ct_ref, conf_ref, loc_ref, prior_ref, scores_ref, pts_ref):
    conf_t = ct_ref[0]
    conf = conf_ref[0]
    mx = jnp.max(conf, axis=0, keepdims=True)
    e = jnp.exp(conf - mx)
    z = jnp.sum(e, axis=0, keepdims=True)
    probs = e / z
    fg = probs[1:, :]
    pos = jax.lax.broadcasted_iota(jnp.int32, (20, _PP), 1)
    valid = (fg >= conf_t) & (pos < _P)
    scores_ref[0] = jnp.where(valid, fg, _NEG)

    l = loc_ref[0]
    lx, ly, lw, lh = l[0:1], l[1:2], l[2:3], l[3:4]
    pcx, pcy, pw, ph = (prior_ref[0:1], prior_ref[1:2],
                        prior_ref[2:3], prior_ref[3:4])
    cx = pcx + lx * (_VAR0) * pw
    cy = pcy + ly * (_VAR0) * ph
    w = pw * jnp.exp(lw * _VAR1)
    h = ph * jnp.exp(lh * _VAR1)
    x1 = cx - w * 0.5
    y1 = cy - h * 0.5
    x2 = cx + w * 0.5
    y2 = cy + h * 0.5
    pts_ref[0, 0:1, :] = x1
    pts_ref[0, 1:2, :] = y1
    pts_ref[0, 2:3, :] = x2
    pts_ref[0, 3:4, :] = y2
    pts_ref[0, 4:5, :] = cx
    pts_ref[0, 5:6, :] = cy
    pts_ref[0, 6:7, :] = w
    pts_ref[0, 7:8, :] = h


def _nms_kernel(nt_ref, scores_ref, pts_ref, out_ref):
    nms_t = nt_ref[0]
    x1a = pts_ref[0, 0]
    y1a = pts_ref[0, 1]
    x2a = pts_ref[0, 2]
    y2a = pts_ref[0, 3]
    area2 = (jnp.maximum(x2a - x1a, 0.0) * jnp.maximum(y2a - y1a, 0.0))

    iota = (jax.lax.broadcasted_iota(jnp.int32, (_ROWS, _LANES), 0) * _LANES
            + jax.lax.broadcasted_iota(jnp.int32, (_ROWS, _LANES), 1))
    lane = jax.lax.broadcasted_iota(jnp.int32, (1, _LANES), 1)
    row8 = jax.lax.broadcasted_iota(jnp.int32, (1, 8), 1)

    def body(t, s):
        m = jnp.max(s)
        sel = m > _NEG
        idx = jnp.min(jnp.where(s == m, iota, jnp.int32(2 ** 30)))
        r = idx // _LANES
        c = idx % _LANES
        lmask = lane == c

        def pick(k):
            row = pts_ref[0, k, pl.ds(r, 1), :]
            return jnp.sum(jnp.where(lmask, row, 0.0))

        x1s = pick(0)
        y1s = pick(1)
        x2s = pick(2)
        y2s = pick(3)
        cxs = pick(4)
        cys = pick(5)
        ws = pick(6)
        hs = pick(7)

        iw = jnp.maximum(jnp.minimum(x2s, x2a) - jnp.maximum(x1s, x1a), 0.0)
        ih = jnp.maximum(jnp.minimum(y2s, y2a) - jnp.maximum(y1s, y1a), 0.0)
        inter = iw * ih
        area1 = jnp.maximum(x2s - x1s, 0.0) * jnp.maximum(y2s - y1s, 0.0)
        union = area1 + area2 - inter
        supp = inter > nms_t * jnp.maximum(union, 1e-12)
        supp = supp | (iota == idx)
        new_s = jnp.where(sel & supp, _NEG, s)

        zero = jnp.float32(0.0)
        row_out = jnp.where(row8 == 0, jnp.where(sel, m, zero),
                  jnp.where(row8 == 1, jnp.where(sel, cxs, zero),
                  jnp.where(row8 == 2, jnp.where(sel, cys, zero),
                  jnp.where(row8 == 3, jnp.where(sel, ws, zero),
                  jnp.where(row8 == 4, jnp.where(sel, hs, zero), zero)))))
        out_ref[0, 0, pl.ds(t, 1), :] = row_out
        return new_s

    s0 = scores_ref[0, 0]
    jax.lax.fori_loop(0, _TOP_K, body, s0)


@jax.jit
def kernel(loc_data, conf_data, prior_data, conf_thresh, nms_thresh):
    B = loc_data.shape[0]
    C = conf_data.shape[1] - 1

    conf_p = jnp.pad(conf_data, ((0, 0), (0, 0), (0, _PP - _P)))
    loc_p = jnp.pad(loc_data, ((0, 0), (0, 0), (0, _PP - _P)))
    prior_p = jnp.pad(prior_data.T, ((0, 0), (0, _PP - _P)))

    scores, pts = pl.pallas_call(
        _prep_kernel,
        grid=(B,),
        in_specs=[
            pl.BlockSpec(memory_space=pltpu.SMEM),
            pl.BlockSpec((1, C + 1, _PP), lambda b: (b, 0, 0)),
            pl.BlockSpec((1, 4, _PP), lambda b: (b, 0, 0)),
            pl.BlockSpec((4, _PP), lambda b: (0, 0)),
        ],
        out_specs=[
            pl.BlockSpec((1, C, _PP), lambda b: (b, 0, 0)),
            pl.BlockSpec((1, 8, _PP), lambda b: (b, 0, 0)),
        ],
        out_shape=[
            jax.ShapeDtypeStruct((B, C, _PP), jnp.float32),
            jax.ShapeDtypeStruct((B, 8, _PP), jnp.float32),
        ],
    )(conf_thresh.reshape(1), conf_p, loc_p, prior_p)

    scores4 = scores.reshape(B, C, _ROWS, _LANES)
    pts4 = pts.reshape(B, 8, _ROWS, _LANES)

    out = pl.pallas_call(
        _nms_kernel,
        grid=(B, C),
        in_specs=[
            pl.BlockSpec(memory_space=pltpu.SMEM),
            pl.BlockSpec((1, 1, _ROWS, _LANES), lambda b, c: (b, c, 0, 0)),
            pl.BlockSpec((1, 8, _ROWS, _LANES), lambda b, c: (b, 0, 0, 0)),
        ],
        out_specs=pl.BlockSpec((1, 1, _TOP_K, 8), lambda b, c: (b, c, 0, 0)),
        out_shape=jax.ShapeDtypeStruct((B, C, _TOP_K, 8), jnp.float32),
    )(nms_thresh.reshape(1), scores4, pts4)

    return out[..., :5]
